# baseline (speedup 1.0000x reference)
import jax
import jax.numpy as jnp
from jax.experimental import pallas as pl
from jax.experimental.pallas import tpu as pltpu

D_MODEL = 1024
N_HEADS = 16
D_HEAD = 64
RANK = 16
DECAY = 0.95
ALPHA_MAX = 0.05
BETA = 0.01

TP = 256
CH = 256
UN = 16

BF = jnp.bfloat16


def _proj_kernel(x_ref, wqkv_ref, wa_ref, ba_ref, gs_ref,
                 q_ref, k_ref, v_ref, al_ref):
    x = x_ref[...]
    res = jnp.dot(x, wqkv_ref[...], preferred_element_type=jnp.float32)
    q_ref[...] = res[:, 0:D_MODEL].astype(BF)
    k_ref[...] = res[:, D_MODEL:2 * D_MODEL].astype(BF)
    v_ref[...] = res[:, 2 * D_MODEL:3 * D_MODEL].astype(BF)
    aa = jnp.dot(x, wa_ref[...], preferred_element_type=jnp.float32) + ba_ref[...]
    al = jax.nn.sigmoid(aa) * gs_ref[...]
    al_ref[...] = jnp.minimum(al, ALPHA_MAX)


def _mm_kernel(x_ref, w_ref, o_ref):
    o_ref[...] = jnp.dot(x_ref[...], w_ref[...],
                         preferred_element_type=jnp.float32)


def _make16(rows):
    pieces = []
    for r in rows:
        pieces.extend([r[:, 256 * t2:256 * (t2 + 1)] for t2 in range(4)])
    return jnp.concatenate(pieces, axis=0)


def _softdecay(x):
    return x - BF(1.0 - DECAY) * x


def _scan_kernel(q_ref, k_ref, v_ref, asel_ref, s4_ref, g4_ref,
                 y_ref, adt_sc, u_sc, w_sc):
    jc = pl.program_id(0)

    @pl.when(jc == 0)
    def _():
        u_sc[...] = jnp.zeros_like(u_sc)
        w_sc[...] = jnp.zeros_like(w_sc)

    adT = jnp.dot(asel_ref[...], s4_ref[...],
                  preferred_element_type=jnp.float32)
    adt_sc[...] = adT.astype(BF).reshape(CH, 16, 256)
    g4 = g4_ref[...]
    inv = 0.125
    beta = BF(BETA)
    one = BF(1.0)

    def sub(ci, _):
        U = u_sc[...]
        W = w_sc[...]
        ys = [[], [], [], []]
        base = pl.multiple_of(ci * UN, UN)
        k_sl = [k_ref[b, pl.ds(base, UN), :] for b in range(4)]
        q_sl = [q_ref[b, pl.ds(base, UN), :] for b in range(4)]
        v_sl = [v_ref[b, pl.ds(base, UN), :] for b in range(4)]
        for ii in range(UN):
            t = base + ii
            k16 = _make16([k_sl[b][ii:ii + 1, :] for b in range(4)])
            q16 = _make16([q_sl[b][ii:ii + 1, :] for b in range(4)])
            v16 = _make16([v_sl[b][ii:ii + 1, :] for b in range(4)])
            a16 = adt_sc[pl.ds(t, 1)].reshape(16, 256)
            k256 = pltpu.repeat(k16, 16, axis=0)
            P = k256 * U
            kub = jnp.concatenate(
                [jnp.dot(P[64 * i4:64 * (i4 + 1)], g4,
                         preferred_element_type=jnp.float32).astype(BF)
                 for i4 in range(4)],
                axis=0)
            tm = pltpu.repeat(a16, 16, axis=0) * kub
            Un = _softdecay(U + k256 * tm)
            Wn = _softdecay(W + pltpu.repeat(v16, 16, axis=0) * tm)
            U = Un - beta * jnp.clip(Un, -one, one)
            W = Wn - beta * jnp.clip(Wn, -one, one)
            z = U * W
            zs = z[0:128] + z[128:256]
            zs = zs[0:64] + zs[64:128]
            zs = zs[0:32] + zs[32:64]
            kf16 = zs[0:16] + zs[16:32]
            sc = jnp.dot(jnp.concatenate([q16 * k16, q16 * kf16], axis=0), g4,
                         preferred_element_type=jnp.float32)
            mix1 = jax.nn.sigmoid((sc[16:32] - sc[0:16]) * inv).astype(BF)
            y16 = v16 + mix1 * (kf16 - v16)
            for b in range(4):
                row = jnp.concatenate(
                    [y16[b * 4 + t2:b * 4 + t2 + 1, :] for t2 in range(4)],
                    axis=1)
                ys[b].append(row)
        u_sc[...] = U
        w_sc[...] = W
        for b in range(4):
            y_ref[b, pl.ds(base, UN), :] = jnp.concatenate(ys[b], axis=0)
        return ()

    jax.lax.fori_loop(0, CH // UN, sub, ())


def _run_mm(x, w, nblk):
    tp = x.shape[0] // nblk
    d_out = w.shape[1]
    return pl.pallas_call(
        _mm_kernel,
        grid=(nblk,),
        in_specs=[
            pl.BlockSpec((tp, x.shape[1]), lambda i: (i, 0)),
            pl.BlockSpec((w.shape[0], d_out), lambda i: (0, 0)),
        ],
        out_specs=pl.BlockSpec((tp, d_out), lambda i: (i, 0)),
        out_shape=jax.ShapeDtypeStruct((x.shape[0], d_out), jnp.float32),
        compiler_params=pltpu.CompilerParams(
            dimension_semantics=("arbitrary",),
        ),
        name="wo_matmul",
    )(x, w)


def kernel(hidden_states, m_gate, alpha_scale, Wq, Wk, Wv, Wo, Wa, ba, mix_logit):
    B, T, D = hidden_states.shape
    BT = B * T
    x = hidden_states.reshape(BT, D)
    wqkv = jnp.concatenate([Wq.T, Wk.T, Wv.T], axis=1)
    gs = (m_gate[..., None] * alpha_scale).reshape(BT, N_HEADS)
    ba2 = ba.reshape(1, N_HEADS)

    nb = BT // TP
    q, k, v, al = pl.pallas_call(
        _proj_kernel,
        grid=(nb,),
        in_specs=[
            pl.BlockSpec((TP, D), lambda i: (i, 0)),
            pl.BlockSpec((D, 3 * D), lambda i: (0, 0)),
            pl.BlockSpec((D, N_HEADS), lambda i: (0, 0)),
            pl.BlockSpec((1, N_HEADS), lambda i: (0, 0)),
            pl.BlockSpec((TP, N_HEADS), lambda i: (i, 0)),
        ],
        out_specs=[
            pl.BlockSpec((TP, D), lambda i: (i, 0)),
            pl.BlockSpec((TP, D), lambda i: (i, 0)),
            pl.BlockSpec((TP, D), lambda i: (i, 0)),
            pl.BlockSpec((TP, N_HEADS), lambda i: (i, 0)),
        ],
        out_shape=[
            jax.ShapeDtypeStruct((BT, D), BF),
            jax.ShapeDtypeStruct((BT, D), BF),
            jax.ShapeDtypeStruct((BT, D), BF),
            jax.ShapeDtypeStruct((BT, N_HEADS), jnp.float32),
        ],
        compiler_params=pltpu.CompilerParams(
            dimension_semantics=("arbitrary",),
        ),
        name="qkv_alpha_proj",
    )(x, wqkv, Wa.T, ba2, gs)

    q4 = q.reshape(B, T, D)
    k4 = k.reshape(B, T, D)
    v4 = v.reshape(B, T, D)
    asel = al.reshape(B, T, 4, 4).transpose(1, 0, 2, 3).reshape(T * 16, 4)

    s4 = jnp.repeat(jnp.eye(4, dtype=jnp.float32), D_HEAD, axis=1)
    g4 = jnp.kron(jnp.eye(4, dtype=BF),
                  jnp.ones((D_HEAD, D_HEAD), BF))

    Y = pl.pallas_call(
        _scan_kernel,
        grid=(T // CH,),
        in_specs=[
            pl.BlockSpec((B, CH, D), lambda t: (0, t, 0)),
            pl.BlockSpec((B, CH, D), lambda t: (0, t, 0)),
            pl.BlockSpec((B, CH, D), lambda t: (0, t, 0)),
            pl.BlockSpec((CH * 16, 4), lambda t: (t, 0)),
            pl.BlockSpec((4, 256), lambda t: (0, 0)),
            pl.BlockSpec((256, 256), lambda t: (0, 0)),
        ],
        out_specs=pl.BlockSpec((B, CH, D), lambda t: (0, t, 0)),
        out_shape=jax.ShapeDtypeStruct((B, T, D), BF),
        scratch_shapes=[
            pltpu.VMEM((CH, 16, 256), BF),
            pltpu.VMEM((256, 256), BF),
            pltpu.VMEM((256, 256), BF),
        ],
        compiler_params=pltpu.CompilerParams(
            dimension_semantics=("arbitrary",),
        ),
        name="fast_weight_scan",
    )(q4, k4, v4, asel, s4, g4)

    y = Y.reshape(BT, D)
    out = _run_mm(y, Wo.T, nb)
    return out.reshape(B, T, D)


# speedup vs baseline: 1.0533x; 1.0533x over previous
import jax
import jax.numpy as jnp
from jax.experimental import pallas as pl
from jax.experimental.pallas import tpu as pltpu

D_MODEL = 1024
N_HEADS = 16
D_HEAD = 64
RANK = 16
DECAY = 0.95
ALPHA_MAX = 0.05
BETA = 0.01

TP = 256
CH = 256
UN = 16

BF = jnp.bfloat16


def _dot_t(a, b):
    return jax.lax.dot_general(a, b, (((1,), (1,)), ((), ())),
                               preferred_element_type=jnp.float32)


def _proj_kernel(x_ref, wq_ref, wk_ref, wv_ref, wa_ref, ba_ref, gs_ref,
                 q_ref, k_ref, v_ref, al_ref):
    x = x_ref[...]
    q_ref[...] = _dot_t(x, wq_ref[...]).astype(BF)
    k_ref[...] = _dot_t(x, wk_ref[...]).astype(BF)
    v_ref[...] = _dot_t(x, wv_ref[...]).astype(BF)
    aa = _dot_t(x, wa_ref[...]) + ba_ref[...]
    al = jax.nn.sigmoid(aa) * gs_ref[...]
    al_ref[...] = jnp.minimum(al, ALPHA_MAX)


def _mm_kernel(x_ref, w_ref, o_ref):
    o_ref[...] = _dot_t(x_ref[...], w_ref[...])


def _make16(rows):
    pieces = []
    for r in rows:
        pieces.extend([r[:, 256 * t2:256 * (t2 + 1)] for t2 in range(4)])
    return jnp.concatenate(pieces, axis=0)


def _softdecay(x):
    return x - BF(1.0 - DECAY) * x


def _scan_kernel(q_ref, k_ref, v_ref, asel_ref, s4_ref, g4_ref,
                 y_ref, adt_sc, u_sc, w_sc):
    jc = pl.program_id(0)

    @pl.when(jc == 0)
    def _():
        u_sc[...] = jnp.zeros_like(u_sc)
        w_sc[...] = jnp.zeros_like(w_sc)

    adT = jnp.dot(asel_ref[...], s4_ref[...],
                  preferred_element_type=jnp.float32)
    adt_sc[...] = adT.astype(BF).reshape(CH, 16, 256)
    g4 = g4_ref[...]
    inv = 0.125
    beta = BF(BETA)
    one = BF(1.0)

    def sub(ci, _):
        U = u_sc[...]
        W = w_sc[...]
        ys = [[], [], [], []]
        base = pl.multiple_of(ci * UN, UN)
        k_sl = [k_ref[b, pl.ds(base, UN), :] for b in range(4)]
        q_sl = [q_ref[b, pl.ds(base, UN), :] for b in range(4)]
        v_sl = [v_ref[b, pl.ds(base, UN), :] for b in range(4)]
        for ii in range(UN):
            t = base + ii
            k16 = _make16([k_sl[b][ii:ii + 1, :] for b in range(4)])
            q16 = _make16([q_sl[b][ii:ii + 1, :] for b in range(4)])
            v16 = _make16([v_sl[b][ii:ii + 1, :] for b in range(4)])
            a16 = adt_sc[pl.ds(t, 1)].reshape(16, 256)
            k256 = pltpu.repeat(k16, 16, axis=0)
            P = k256 * U
            kub = jnp.concatenate(
                [jnp.dot(P[0:128], g4,
                         preferred_element_type=jnp.float32).astype(BF),
                 jnp.dot(P[128:256], g4,
                         preferred_element_type=jnp.float32).astype(BF)],
                axis=0)
            tm = pltpu.repeat(a16, 16, axis=0) * kub
            Un = _softdecay(U + k256 * tm)
            Wn = _softdecay(W + pltpu.repeat(v16, 16, axis=0) * tm)
            U = Un - beta * jnp.clip(Un, -one, one)
            W = Wn - beta * jnp.clip(Wn, -one, one)
            z = U * W
            zs = z[0:128] + z[128:256]
            zs = zs[0:64] + zs[64:128]
            zs = zs[0:32] + zs[32:64]
            kf16 = zs[0:16] + zs[16:32]
            sc = jnp.dot(jnp.concatenate([q16 * k16, q16 * kf16], axis=0), g4,
                         preferred_element_type=jnp.float32)
            mix1 = jax.nn.sigmoid((sc[16:32] - sc[0:16]) * inv).astype(BF)
            y16 = v16 + mix1 * (kf16 - v16)
            for b in range(4):
                row = jnp.concatenate(
                    [y16[b * 4 + t2:b * 4 + t2 + 1, :] for t2 in range(4)],
                    axis=1)
                ys[b].append(row)
        u_sc[...] = U
        w_sc[...] = W
        for b in range(4):
            y_ref[b, pl.ds(base, UN), :] = jnp.concatenate(ys[b], axis=0)
        return ()

    jax.lax.fori_loop(0, CH // UN, sub, ())


def _run_mm(x, w, nblk):
    tp = x.shape[0] // nblk
    d_out = w.shape[1]
    return pl.pallas_call(
        _mm_kernel,
        grid=(nblk,),
        in_specs=[
            pl.BlockSpec((tp, x.shape[1]), lambda i: (i, 0)),
            pl.BlockSpec((w.shape[0], d_out), lambda i: (0, 0)),
        ],
        out_specs=pl.BlockSpec((tp, d_out), lambda i: (i, 0)),
        out_shape=jax.ShapeDtypeStruct((x.shape[0], d_out), jnp.float32),
        compiler_params=pltpu.CompilerParams(
            dimension_semantics=("arbitrary",),
        ),
        name="wo_matmul",
    )(x, w)


def kernel(hidden_states, m_gate, alpha_scale, Wq, Wk, Wv, Wo, Wa, ba, mix_logit):
    B, T, D = hidden_states.shape
    BT = B * T
    x = hidden_states.reshape(BT, D)
    gs = (m_gate[..., None] * alpha_scale).reshape(BT, N_HEADS)
    ba2 = ba.reshape(1, N_HEADS)

    nb = BT // TP
    q, k, v, al = pl.pallas_call(
        _proj_kernel,
        grid=(nb,),
        in_specs=[
            pl.BlockSpec((TP, D), lambda i: (i, 0)),
            pl.BlockSpec((D, D), lambda i: (0, 0)),
            pl.BlockSpec((D, D), lambda i: (0, 0)),
            pl.BlockSpec((D, D), lambda i: (0, 0)),
            pl.BlockSpec((N_HEADS, D), lambda i: (0, 0)),
            pl.BlockSpec((1, N_HEADS), lambda i: (0, 0)),
            pl.BlockSpec((TP, N_HEADS), lambda i: (i, 0)),
        ],
        out_specs=[
            pl.BlockSpec((TP, D), lambda i: (i, 0)),
            pl.BlockSpec((TP, D), lambda i: (i, 0)),
            pl.BlockSpec((TP, D), lambda i: (i, 0)),
            pl.BlockSpec((TP, N_HEADS), lambda i: (i, 0)),
        ],
        out_shape=[
            jax.ShapeDtypeStruct((BT, D), BF),
            jax.ShapeDtypeStruct((BT, D), BF),
            jax.ShapeDtypeStruct((BT, D), BF),
            jax.ShapeDtypeStruct((BT, N_HEADS), jnp.float32),
        ],
        compiler_params=pltpu.CompilerParams(
            dimension_semantics=("arbitrary",),
        ),
        name="qkv_alpha_proj",
    )(x, Wq, Wk, Wv, Wa, ba2, gs)

    q4 = q.reshape(B, T, D)
    k4 = k.reshape(B, T, D)
    v4 = v.reshape(B, T, D)
    asel = al.reshape(B, T, 4, 4).transpose(1, 0, 2, 3).reshape(T * 16, 4)

    s4 = jnp.repeat(jnp.eye(4, dtype=jnp.float32), D_HEAD, axis=1)
    g4 = jnp.kron(jnp.eye(4, dtype=BF),
                  jnp.ones((D_HEAD, D_HEAD), BF))

    Y = pl.pallas_call(
        _scan_kernel,
        grid=(T // CH,),
        in_specs=[
            pl.BlockSpec((B, CH, D), lambda t: (0, t, 0)),
            pl.BlockSpec((B, CH, D), lambda t: (0, t, 0)),
            pl.BlockSpec((B, CH, D), lambda t: (0, t, 0)),
            pl.BlockSpec((CH * 16, 4), lambda t: (t, 0)),
            pl.BlockSpec((4, 256), lambda t: (0, 0)),
            pl.BlockSpec((256, 256), lambda t: (0, 0)),
        ],
        out_specs=pl.BlockSpec((B, CH, D), lambda t: (0, t, 0)),
        out_shape=jax.ShapeDtypeStruct((B, T, D), BF),
        scratch_shapes=[
            pltpu.VMEM((CH, 16, 256), BF),
            pltpu.VMEM((256, 256), BF),
            pltpu.VMEM((256, 256), BF),
        ],
        compiler_params=pltpu.CompilerParams(
            dimension_semantics=("arbitrary",),
        ),
        name="fast_weight_scan",
    )(q4, k4, v4, asel, s4, g4)

    y = Y.reshape(BT, D)
    out = _run_mm(y, Wo, nb)
    return out.reshape(B, T, D)


# speedup vs baseline: 1.0691x; 1.0150x over previous
import jax
import jax.numpy as jnp
from jax.experimental import pallas as pl
from jax.experimental.pallas import tpu as pltpu

D_MODEL = 1024
N_HEADS = 16
D_HEAD = 64
RANK = 16
DECAY = 0.95
ALPHA_MAX = 0.05
BETA = 0.01

TP = 256
CH = 256
UN = 32

BF = jnp.bfloat16


def _dot_t(a, b):
    return jax.lax.dot_general(a, b, (((1,), (1,)), ((), ())),
                               preferred_element_type=jnp.float32)


def _proj_kernel(x_ref, wq_ref, wk_ref, wv_ref, wa_ref, ba_ref, gs_ref,
                 q_ref, k_ref, v_ref, al_ref):
    x = x_ref[...]
    q_ref[...] = _dot_t(x, wq_ref[...]).astype(BF)
    k_ref[...] = _dot_t(x, wk_ref[...]).astype(BF)
    v_ref[...] = _dot_t(x, wv_ref[...]).astype(BF)
    aa = _dot_t(x, wa_ref[...]) + ba_ref[...]
    al = jax.nn.sigmoid(aa) * gs_ref[...]
    al_ref[...] = jnp.minimum(al, ALPHA_MAX)


def _mm_kernel(x_ref, w_ref, o_ref):
    o_ref[...] = _dot_t(x_ref[...], w_ref[...])


def _make16(rows):
    pieces = []
    for r in rows:
        pieces.extend([r[:, 256 * t2:256 * (t2 + 1)] for t2 in range(4)])
    return jnp.concatenate(pieces, axis=0)


def _softdecay(x):
    return x - BF(1.0 - DECAY) * x


def _scan_kernel(q_ref, k_ref, v_ref, asel_ref, s4_ref, g4_ref,
                 y_ref, adt_sc, u_sc, w_sc):
    jc = pl.program_id(0)

    @pl.when(jc == 0)
    def _():
        u_sc[...] = jnp.zeros_like(u_sc)
        w_sc[...] = jnp.zeros_like(w_sc)

    adT = jnp.dot(asel_ref[...], s4_ref[...],
                  preferred_element_type=jnp.float32)
    adt_sc[...] = adT.astype(BF).reshape(CH, 16, 256)
    g4 = g4_ref[...]
    inv = 0.125
    beta = BF(BETA)
    one = BF(1.0)

    def sub(ci, _):
        U = u_sc[...]
        W = w_sc[...]
        ys = [[], [], [], []]
        base = pl.multiple_of(ci * UN, UN)
        k_sl = [k_ref[b, pl.ds(base, UN), :] for b in range(4)]
        q_sl = [q_ref[b, pl.ds(base, UN), :] for b in range(4)]
        v_sl = [v_ref[b, pl.ds(base, UN), :] for b in range(4)]
        for ii in range(UN):
            t = base + ii
            k16 = _make16([k_sl[b][ii:ii + 1, :] for b in range(4)])
            q16 = _make16([q_sl[b][ii:ii + 1, :] for b in range(4)])
            v16 = _make16([v_sl[b][ii:ii + 1, :] for b in range(4)])
            a16 = adt_sc[pl.ds(t, 1)].reshape(16, 256)
            k256 = pltpu.repeat(k16, 16, axis=0)
            P = k256 * U
            kub = jnp.concatenate(
                [jnp.dot(P[0:128], g4,
                         preferred_element_type=jnp.float32).astype(BF),
                 jnp.dot(P[128:256], g4,
                         preferred_element_type=jnp.float32).astype(BF)],
                axis=0)
            tm = pltpu.repeat(a16, 16, axis=0) * kub
            Un = _softdecay(U + k256 * tm)
            Wn = _softdecay(W + pltpu.repeat(v16, 16, axis=0) * tm)
            U = Un - beta * jnp.clip(Un, -one, one)
            W = Wn - beta * jnp.clip(Wn, -one, one)
            z = U * W
            zs = z[0:128] + z[128:256]
            zs = zs[0:64] + zs[64:128]
            zs = zs[0:32] + zs[32:64]
            kf16 = zs[0:16] + zs[16:32]
            sc = jnp.dot(jnp.concatenate([q16 * k16, q16 * kf16], axis=0), g4,
                         preferred_element_type=jnp.float32)
            mix1 = jax.nn.sigmoid((sc[16:32] - sc[0:16]) * inv).astype(BF)
            y16 = v16 + mix1 * (kf16 - v16)
            for b in range(4):
                row = jnp.concatenate(
                    [y16[b * 4 + t2:b * 4 + t2 + 1, :] for t2 in range(4)],
                    axis=1)
                ys[b].append(row)
        u_sc[...] = U
        w_sc[...] = W
        for b in range(4):
            y_ref[b, pl.ds(base, UN), :] = jnp.concatenate(ys[b], axis=0)
        return ()

    jax.lax.fori_loop(0, CH // UN, sub, ())


def _run_mm(x, w, nblk):
    tp = x.shape[0] // nblk
    d_out = w.shape[1]
    return pl.pallas_call(
        _mm_kernel,
        grid=(nblk,),
        in_specs=[
            pl.BlockSpec((tp, x.shape[1]), lambda i: (i, 0)),
            pl.BlockSpec((w.shape[0], d_out), lambda i: (0, 0)),
        ],
        out_specs=pl.BlockSpec((tp, d_out), lambda i: (i, 0)),
        out_shape=jax.ShapeDtypeStruct((x.shape[0], d_out), jnp.float32),
        compiler_params=pltpu.CompilerParams(
            dimension_semantics=("arbitrary",),
        ),
        name="wo_matmul",
    )(x, w)


def kernel(hidden_states, m_gate, alpha_scale, Wq, Wk, Wv, Wo, Wa, ba, mix_logit):
    B, T, D = hidden_states.shape
    BT = B * T
    x = hidden_states.reshape(BT, D)
    gs = (m_gate[..., None] * alpha_scale).reshape(BT, N_HEADS)
    ba2 = ba.reshape(1, N_HEADS)

    nb = BT // TP
    q, k, v, al = pl.pallas_call(
        _proj_kernel,
        grid=(nb,),
        in_specs=[
            pl.BlockSpec((TP, D), lambda i: (i, 0)),
            pl.BlockSpec((D, D), lambda i: (0, 0)),
            pl.BlockSpec((D, D), lambda i: (0, 0)),
            pl.BlockSpec((D, D), lambda i: (0, 0)),
            pl.BlockSpec((N_HEADS, D), lambda i: (0, 0)),
            pl.BlockSpec((1, N_HEADS), lambda i: (0, 0)),
            pl.BlockSpec((TP, N_HEADS), lambda i: (i, 0)),
        ],
        out_specs=[
            pl.BlockSpec((TP, D), lambda i: (i, 0)),
            pl.BlockSpec((TP, D), lambda i: (i, 0)),
            pl.BlockSpec((TP, D), lambda i: (i, 0)),
            pl.BlockSpec((TP, N_HEADS), lambda i: (i, 0)),
        ],
        out_shape=[
            jax.ShapeDtypeStruct((BT, D), BF),
            jax.ShapeDtypeStruct((BT, D), BF),
            jax.ShapeDtypeStruct((BT, D), BF),
            jax.ShapeDtypeStruct((BT, N_HEADS), jnp.float32),
        ],
        compiler_params=pltpu.CompilerParams(
            dimension_semantics=("arbitrary",),
        ),
        name="qkv_alpha_proj",
    )(x, Wq, Wk, Wv, Wa, ba2, gs)

    q4 = q.reshape(B, T, D)
    k4 = k.reshape(B, T, D)
    v4 = v.reshape(B, T, D)
    asel = al.reshape(B, T, 4, 4).transpose(1, 0, 2, 3).reshape(T * 16, 4)

    s4 = jnp.repeat(jnp.eye(4, dtype=jnp.float32), D_HEAD, axis=1)
    g4 = jnp.kron(jnp.eye(4, dtype=BF),
                  jnp.ones((D_HEAD, D_HEAD), BF))

    Y = pl.pallas_call(
        _scan_kernel,
        grid=(T // CH,),
        in_specs=[
            pl.BlockSpec((B, CH, D), lambda t: (0, t, 0)),
            pl.BlockSpec((B, CH, D), lambda t: (0, t, 0)),
            pl.BlockSpec((B, CH, D), lambda t: (0, t, 0)),
            pl.BlockSpec((CH * 16, 4), lambda t: (t, 0)),
            pl.BlockSpec((4, 256), lambda t: (0, 0)),
            pl.BlockSpec((256, 256), lambda t: (0, 0)),
        ],
        out_specs=pl.BlockSpec((B, CH, D), lambda t: (0, t, 0)),
        out_shape=jax.ShapeDtypeStruct((B, T, D), BF),
        scratch_shapes=[
            pltpu.VMEM((CH, 16, 256), BF),
            pltpu.VMEM((256, 256), BF),
            pltpu.VMEM((256, 256), BF),
        ],
        compiler_params=pltpu.CompilerParams(
            dimension_semantics=("arbitrary",),
        ),
        name="fast_weight_scan",
    )(q4, k4, v4, asel, s4, g4)

    y = Y.reshape(BT, D)
    out = _run_mm(y, Wo, nb)
    return out.reshape(B, T, D)


# speedup vs baseline: 1.0709x; 1.0017x over previous
import jax
import jax.numpy as jnp
from jax.experimental import pallas as pl
from jax.experimental.pallas import tpu as pltpu

D_MODEL = 1024
N_HEADS = 16
D_HEAD = 64
RANK = 16
DECAY = 0.95
ALPHA_MAX = 0.05
BETA = 0.01

TP = 256
CH = 512
UN = 32

BF = jnp.bfloat16


def _dot_t(a, b):
    return jax.lax.dot_general(a, b, (((1,), (1,)), ((), ())),
                               preferred_element_type=jnp.float32)


def _proj_kernel(x_ref, wq_ref, wk_ref, wv_ref, wa_ref, ba_ref, gs_ref,
                 q_ref, k_ref, v_ref, al_ref):
    x = x_ref[...]
    q_ref[...] = _dot_t(x, wq_ref[...]).astype(BF)
    k_ref[...] = _dot_t(x, wk_ref[...]).astype(BF)
    v_ref[...] = _dot_t(x, wv_ref[...]).astype(BF)
    aa = _dot_t(x, wa_ref[...]) + ba_ref[...]
    al = jax.nn.sigmoid(aa) * gs_ref[...]
    al_ref[...] = jnp.minimum(al, ALPHA_MAX)


def _mm_kernel(x_ref, w_ref, o_ref):
    o_ref[...] = _dot_t(x_ref[...], w_ref[...])


def _make16(rows):
    pieces = []
    for r in rows:
        pieces.extend([r[:, 256 * t2:256 * (t2 + 1)] for t2 in range(4)])
    return jnp.concatenate(pieces, axis=0)


def _softdecay(x):
    return x - BF(1.0 - DECAY) * x


def _scan_kernel(q_ref, k_ref, v_ref, asel_ref, s4_ref, g4_ref,
                 y_ref, adt_sc, u_sc, w_sc):
    jc = pl.program_id(0)

    @pl.when(jc == 0)
    def _():
        u_sc[...] = jnp.zeros_like(u_sc)
        w_sc[...] = jnp.zeros_like(w_sc)

    adT = jnp.dot(asel_ref[...], s4_ref[...],
                  preferred_element_type=jnp.float32)
    adt_sc[...] = adT.astype(BF).reshape(CH, 16, 256)
    g4 = g4_ref[...]
    inv = 0.125
    beta = BF(BETA)
    one = BF(1.0)

    def sub(ci, _):
        U = u_sc[...]
        W = w_sc[...]
        ys = [[], [], [], []]
        base = pl.multiple_of(ci * UN, UN)
        k_sl = [k_ref[b, pl.ds(base, UN), :] for b in range(4)]
        q_sl = [q_ref[b, pl.ds(base, UN), :] for b in range(4)]
        v_sl = [v_ref[b, pl.ds(base, UN), :] for b in range(4)]
        for ii in range(UN):
            t = base + ii
            k16 = _make16([k_sl[b][ii:ii + 1, :] for b in range(4)])
            q16 = _make16([q_sl[b][ii:ii + 1, :] for b in range(4)])
            v16 = _make16([v_sl[b][ii:ii + 1, :] for b in range(4)])
            a16 = adt_sc[pl.ds(t, 1)].reshape(16, 256)
            k256 = pltpu.repeat(k16, 16, axis=0)
            P = k256 * U
            kub = jnp.concatenate(
                [jnp.dot(P[0:128], g4,
                         preferred_element_type=jnp.float32).astype(BF),
                 jnp.dot(P[128:256], g4,
                         preferred_element_type=jnp.float32).astype(BF)],
                axis=0)
            tm = pltpu.repeat(a16, 16, axis=0) * kub
            Un = _softdecay(U + k256 * tm)
            Wn = _softdecay(W + pltpu.repeat(v16, 16, axis=0) * tm)
            U = Un - beta * jnp.clip(Un, -one, one)
            W = Wn - beta * jnp.clip(Wn, -one, one)
            z = U * W
            zs = z[0:128] + z[128:256]
            zs = zs[0:64] + zs[64:128]
            zs = zs[0:32] + zs[32:64]
            kf16 = zs[0:16] + zs[16:32]
            sc = jnp.dot(jnp.concatenate([q16 * k16, q16 * kf16], axis=0), g4,
                         preferred_element_type=jnp.float32)
            mix1 = jax.nn.sigmoid((sc[16:32] - sc[0:16]) * inv).astype(BF)
            y16 = v16 + mix1 * (kf16 - v16)
            for b in range(4):
                row = jnp.concatenate(
                    [y16[b * 4 + t2:b * 4 + t2 + 1, :] for t2 in range(4)],
                    axis=1)
                ys[b].append(row)
        u_sc[...] = U
        w_sc[...] = W
        for b in range(4):
            y_ref[b, pl.ds(base, UN), :] = jnp.concatenate(ys[b], axis=0)
        return ()

    jax.lax.fori_loop(0, CH // UN, sub, ())


def _run_mm(x, w, nblk):
    tp = x.shape[0] // nblk
    d_out = w.shape[1]
    return pl.pallas_call(
        _mm_kernel,
        grid=(nblk,),
        in_specs=[
            pl.BlockSpec((tp, x.shape[1]), lambda i: (i, 0)),
            pl.BlockSpec((w.shape[0], d_out), lambda i: (0, 0)),
        ],
        out_specs=pl.BlockSpec((tp, d_out), lambda i: (i, 0)),
        out_shape=jax.ShapeDtypeStruct((x.shape[0], d_out), jnp.float32),
        compiler_params=pltpu.CompilerParams(
            dimension_semantics=("arbitrary",),
        ),
        name="wo_matmul",
    )(x, w)


def kernel(hidden_states, m_gate, alpha_scale, Wq, Wk, Wv, Wo, Wa, ba, mix_logit):
    B, T, D = hidden_states.shape
    BT = B * T
    x = hidden_states.reshape(BT, D)
    gs = (m_gate[..., None] * alpha_scale).reshape(BT, N_HEADS)
    ba2 = ba.reshape(1, N_HEADS)

    nb = BT // TP
    q, k, v, al = pl.pallas_call(
        _proj_kernel,
        grid=(nb,),
        in_specs=[
            pl.BlockSpec((TP, D), lambda i: (i, 0)),
            pl.BlockSpec((D, D), lambda i: (0, 0)),
            pl.BlockSpec((D, D), lambda i: (0, 0)),
            pl.BlockSpec((D, D), lambda i: (0, 0)),
            pl.BlockSpec((N_HEADS, D), lambda i: (0, 0)),
            pl.BlockSpec((1, N_HEADS), lambda i: (0, 0)),
            pl.BlockSpec((TP, N_HEADS), lambda i: (i, 0)),
        ],
        out_specs=[
            pl.BlockSpec((TP, D), lambda i: (i, 0)),
            pl.BlockSpec((TP, D), lambda i: (i, 0)),
            pl.BlockSpec((TP, D), lambda i: (i, 0)),
            pl.BlockSpec((TP, N_HEADS), lambda i: (i, 0)),
        ],
        out_shape=[
            jax.ShapeDtypeStruct((BT, D), BF),
            jax.ShapeDtypeStruct((BT, D), BF),
            jax.ShapeDtypeStruct((BT, D), BF),
            jax.ShapeDtypeStruct((BT, N_HEADS), jnp.float32),
        ],
        compiler_params=pltpu.CompilerParams(
            dimension_semantics=("arbitrary",),
        ),
        name="qkv_alpha_proj",
    )(x, Wq, Wk, Wv, Wa, ba2, gs)

    q4 = q.reshape(B, T, D)
    k4 = k.reshape(B, T, D)
    v4 = v.reshape(B, T, D)
    asel = al.reshape(B, T, 4, 4).transpose(1, 0, 2, 3).reshape(T * 16, 4)

    s4 = jnp.repeat(jnp.eye(4, dtype=jnp.float32), D_HEAD, axis=1)
    g4 = jnp.kron(jnp.eye(4, dtype=BF),
                  jnp.ones((D_HEAD, D_HEAD), BF))

    Y = pl.pallas_call(
        _scan_kernel,
        grid=(T // CH,),
        in_specs=[
            pl.BlockSpec((B, CH, D), lambda t: (0, t, 0)),
            pl.BlockSpec((B, CH, D), lambda t: (0, t, 0)),
            pl.BlockSpec((B, CH, D), lambda t: (0, t, 0)),
            pl.BlockSpec((CH * 16, 4), lambda t: (t, 0)),
            pl.BlockSpec((4, 256), lambda t: (0, 0)),
            pl.BlockSpec((256, 256), lambda t: (0, 0)),
        ],
        out_specs=pl.BlockSpec((B, CH, D), lambda t: (0, t, 0)),
        out_shape=jax.ShapeDtypeStruct((B, T, D), BF),
        scratch_shapes=[
            pltpu.VMEM((CH, 16, 256), BF),
            pltpu.VMEM((256, 256), BF),
            pltpu.VMEM((256, 256), BF),
        ],
        compiler_params=pltpu.CompilerParams(
            dimension_semantics=("arbitrary",),
        ),
        name="fast_weight_scan",
    )(q4, k4, v4, asel, s4, g4)

    y = Y.reshape(BT, D)
    out = _run_mm(y, Wo, nb)
    return out.reshape(B, T, D)


# speedup vs baseline: 1.1031x; 1.0301x over previous
import jax
import jax.numpy as jnp
from jax.experimental import pallas as pl
from jax.experimental.pallas import tpu as pltpu

D_MODEL = 1024
N_HEADS = 16
D_HEAD = 64
RANK = 16
DECAY = 0.95
ALPHA_MAX = 0.05
BETA = 0.01

TP = 256
CH = 256
UN = 32

BF = jnp.bfloat16


def _dot_t(a, b):
    return jax.lax.dot_general(a, b, (((1,), (1,)), ((), ())),
                               preferred_element_type=jnp.float32)


def _proj_kernel(x_ref, wq_ref, wk_ref, wv_ref, wa_ref, ba_ref, gs_ref,
                 q_ref, k_ref, v_ref, al_ref):
    x = x_ref[...]
    q_ref[...] = _dot_t(x, wq_ref[...]).astype(BF)
    k_ref[...] = _dot_t(x, wk_ref[...]).astype(BF)
    v_ref[...] = _dot_t(x, wv_ref[...]).astype(BF)
    aa = _dot_t(x, wa_ref[...]) + ba_ref[...]
    al = jax.nn.sigmoid(aa) * gs_ref[...]
    al_ref[...] = jnp.minimum(al, ALPHA_MAX)


def _make16(rows):
    pieces = []
    for r in rows:
        pieces.extend([r[:, 256 * t2:256 * (t2 + 1)] for t2 in range(4)])
    return jnp.concatenate(pieces, axis=0)


def _softdecay(x):
    return x - BF(1.0 - DECAY) * x


def _scan_kernel(q_ref, k_ref, v_ref, asel_ref, s4_ref, g4_ref, wo_ref,
                 out_ref, y_sc, adt_sc, u_sc, w_sc):
    jc = pl.program_id(0)

    @pl.when(jc == 0)
    def _():
        u_sc[...] = jnp.zeros_like(u_sc)
        w_sc[...] = jnp.zeros_like(w_sc)

    adT = jnp.dot(asel_ref[...], s4_ref[...],
                  preferred_element_type=jnp.float32)
    adt_sc[...] = adT.astype(BF).reshape(CH, 16, 256)
    g4 = g4_ref[...]
    inv = 0.125
    beta = BF(BETA)
    one = BF(1.0)

    def sub(ci, _):
        U = u_sc[...]
        W = w_sc[...]
        ys = [[], [], [], []]
        base = pl.multiple_of(ci * UN, UN)
        k_sl = [k_ref[b, pl.ds(base, UN), :] for b in range(4)]
        q_sl = [q_ref[b, pl.ds(base, UN), :] for b in range(4)]
        v_sl = [v_ref[b, pl.ds(base, UN), :] for b in range(4)]
        for ii in range(UN):
            t = base + ii
            k16 = _make16([k_sl[b][ii:ii + 1, :] for b in range(4)])
            q16 = _make16([q_sl[b][ii:ii + 1, :] for b in range(4)])
            v16 = _make16([v_sl[b][ii:ii + 1, :] for b in range(4)])
            a16 = adt_sc[pl.ds(t, 1)].reshape(16, 256)
            k256 = pltpu.repeat(k16, 16, axis=0)
            P = k256 * U
            kub = jnp.concatenate(
                [jnp.dot(P[0:128], g4,
                         preferred_element_type=jnp.float32).astype(BF),
                 jnp.dot(P[128:256], g4,
                         preferred_element_type=jnp.float32).astype(BF)],
                axis=0)
            tm = pltpu.repeat(a16, 16, axis=0) * kub
            Un = _softdecay(U + k256 * tm)
            Wn = _softdecay(W + pltpu.repeat(v16, 16, axis=0) * tm)
            U = Un - beta * jnp.clip(Un, -one, one)
            W = Wn - beta * jnp.clip(Wn, -one, one)
            z = U * W
            zs = z[0:128] + z[128:256]
            zs = zs[0:64] + zs[64:128]
            zs = zs[0:32] + zs[32:64]
            kf16 = zs[0:16] + zs[16:32]
            sc = jnp.dot(jnp.concatenate([q16 * k16, q16 * kf16], axis=0), g4,
                         preferred_element_type=jnp.float32)
            mix1 = jax.nn.sigmoid((sc[16:32] - sc[0:16]) * inv).astype(BF)
            y16 = v16 + mix1 * (kf16 - v16)
            for b in range(4):
                row = jnp.concatenate(
                    [y16[b * 4 + t2:b * 4 + t2 + 1, :] for t2 in range(4)],
                    axis=1)
                ys[b].append(row)
        u_sc[...] = U
        w_sc[...] = W
        for b in range(4):
            y_sc[b, pl.ds(base, UN), :] = jnp.concatenate(ys[b], axis=0)
        return ()

    jax.lax.fori_loop(0, CH // UN, sub, ())

    for b in range(4):
        out_ref[b] = _dot_t(y_sc[b], wo_ref[...])


def kernel(hidden_states, m_gate, alpha_scale, Wq, Wk, Wv, Wo, Wa, ba, mix_logit):
    B, T, D = hidden_states.shape
    BT = B * T
    x = hidden_states.reshape(BT, D)
    gs = (m_gate[..., None] * alpha_scale).reshape(BT, N_HEADS)
    ba2 = ba.reshape(1, N_HEADS)

    nb = BT // TP
    q, k, v, al = pl.pallas_call(
        _proj_kernel,
        grid=(nb,),
        in_specs=[
            pl.BlockSpec((TP, D), lambda i: (i, 0)),
            pl.BlockSpec((D, D), lambda i: (0, 0)),
            pl.BlockSpec((D, D), lambda i: (0, 0)),
            pl.BlockSpec((D, D), lambda i: (0, 0)),
            pl.BlockSpec((N_HEADS, D), lambda i: (0, 0)),
            pl.BlockSpec((1, N_HEADS), lambda i: (0, 0)),
            pl.BlockSpec((TP, N_HEADS), lambda i: (i, 0)),
        ],
        out_specs=[
            pl.BlockSpec((TP, D), lambda i: (i, 0)),
            pl.BlockSpec((TP, D), lambda i: (i, 0)),
            pl.BlockSpec((TP, D), lambda i: (i, 0)),
            pl.BlockSpec((TP, N_HEADS), lambda i: (i, 0)),
        ],
        out_shape=[
            jax.ShapeDtypeStruct((BT, D), BF),
            jax.ShapeDtypeStruct((BT, D), BF),
            jax.ShapeDtypeStruct((BT, D), BF),
            jax.ShapeDtypeStruct((BT, N_HEADS), jnp.float32),
        ],
        compiler_params=pltpu.CompilerParams(
            dimension_semantics=("arbitrary",),
        ),
        name="qkv_alpha_proj",
    )(x, Wq, Wk, Wv, Wa, ba2, gs)

    q4 = q.reshape(B, T, D)
    k4 = k.reshape(B, T, D)
    v4 = v.reshape(B, T, D)
    asel = al.reshape(B, T, 4, 4).transpose(1, 0, 2, 3).reshape(T * 16, 4)

    s4 = jnp.repeat(jnp.eye(4, dtype=jnp.float32), D_HEAD, axis=1)
    g4 = jnp.kron(jnp.eye(4, dtype=BF),
                  jnp.ones((D_HEAD, D_HEAD), BF))

    out = pl.pallas_call(
        _scan_kernel,
        grid=(T // CH,),
        in_specs=[
            pl.BlockSpec((B, CH, D), lambda t: (0, t, 0)),
            pl.BlockSpec((B, CH, D), lambda t: (0, t, 0)),
            pl.BlockSpec((B, CH, D), lambda t: (0, t, 0)),
            pl.BlockSpec((CH * 16, 4), lambda t: (t, 0)),
            pl.BlockSpec((4, 256), lambda t: (0, 0)),
            pl.BlockSpec((256, 256), lambda t: (0, 0)),
            pl.BlockSpec((D, D), lambda t: (0, 0)),
        ],
        out_specs=pl.BlockSpec((B, CH, D), lambda t: (0, t, 0)),
        out_shape=jax.ShapeDtypeStruct((B, T, D), jnp.float32),
        scratch_shapes=[
            pltpu.VMEM((B, CH, D), BF),
            pltpu.VMEM((CH, 16, 256), BF),
            pltpu.VMEM((256, 256), BF),
            pltpu.VMEM((256, 256), BF),
        ],
        compiler_params=pltpu.CompilerParams(
            dimension_semantics=("arbitrary",),
        ),
        name="fast_weight_scan",
    )(q4, k4, v4, asel, s4, g4, Wo)

    return out


# speedup vs baseline: 1.1103x; 1.0065x over previous
import jax
import jax.numpy as jnp
from jax.experimental import pallas as pl
from jax.experimental.pallas import tpu as pltpu

D_MODEL = 1024
N_HEADS = 16
D_HEAD = 64
RANK = 16
DECAY = 0.95
ALPHA_MAX = 0.05
BETA = 0.01

TP = 256
CH = 256
UN = 64

BF = jnp.bfloat16


def _dot_t(a, b):
    return jax.lax.dot_general(a, b, (((1,), (1,)), ((), ())),
                               preferred_element_type=jnp.float32)


def _proj_kernel(x_ref, wq_ref, wk_ref, wv_ref, wa_ref, ba_ref, gs_ref,
                 q_ref, k_ref, v_ref, al_ref):
    x = x_ref[...]
    q_ref[...] = _dot_t(x, wq_ref[...]).astype(BF)
    k_ref[...] = _dot_t(x, wk_ref[...]).astype(BF)
    v_ref[...] = _dot_t(x, wv_ref[...]).astype(BF)
    aa = _dot_t(x, wa_ref[...]) + ba_ref[...]
    al = jax.nn.sigmoid(aa) * gs_ref[...]
    al_ref[...] = jnp.minimum(al, ALPHA_MAX)


def _make16(rows):
    pieces = []
    for r in rows:
        pieces.extend([r[:, 256 * t2:256 * (t2 + 1)] for t2 in range(4)])
    return jnp.concatenate(pieces, axis=0)


def _softdecay(x):
    return x - BF(1.0 - DECAY) * x


def _scan_kernel(q_ref, k_ref, v_ref, asel_ref, s4_ref, g4_ref, wo_ref,
                 out_ref, y_sc, adt_sc, u_sc, w_sc):
    jc = pl.program_id(0)

    @pl.when(jc == 0)
    def _():
        u_sc[...] = jnp.zeros_like(u_sc)
        w_sc[...] = jnp.zeros_like(w_sc)

    adT = jnp.dot(asel_ref[...], s4_ref[...],
                  preferred_element_type=jnp.float32)
    adt_sc[...] = adT.astype(BF).reshape(CH, 16, 256)
    g4 = g4_ref[...]
    inv = 0.125
    beta = BF(BETA)
    one = BF(1.0)

    def sub(ci, _):
        U = u_sc[...]
        W = w_sc[...]
        ys = [[], [], [], []]
        base = pl.multiple_of(ci * UN, UN)
        k_sl = [k_ref[b, pl.ds(base, UN), :] for b in range(4)]
        q_sl = [q_ref[b, pl.ds(base, UN), :] for b in range(4)]
        v_sl = [v_ref[b, pl.ds(base, UN), :] for b in range(4)]
        for ii in range(UN):
            t = base + ii
            k16 = _make16([k_sl[b][ii:ii + 1, :] for b in range(4)])
            q16 = _make16([q_sl[b][ii:ii + 1, :] for b in range(4)])
            v16 = _make16([v_sl[b][ii:ii + 1, :] for b in range(4)])
            a16 = adt_sc[pl.ds(t, 1)].reshape(16, 256)
            k256 = pltpu.repeat(k16, 16, axis=0)
            P = k256 * U
            kub = jnp.concatenate(
                [jnp.dot(P[0:128], g4,
                         preferred_element_type=jnp.float32).astype(BF),
                 jnp.dot(P[128:256], g4,
                         preferred_element_type=jnp.float32).astype(BF)],
                axis=0)
            tm = pltpu.repeat(a16, 16, axis=0) * kub
            Un = _softdecay(U + k256 * tm)
            Wn = _softdecay(W + pltpu.repeat(v16, 16, axis=0) * tm)
            U = Un - beta * jnp.clip(Un, -one, one)
            W = Wn - beta * jnp.clip(Wn, -one, one)
            z = U * W
            zs = z[0:128] + z[128:256]
            zs = zs[0:64] + zs[64:128]
            zs = zs[0:32] + zs[32:64]
            kf16 = zs[0:16] + zs[16:32]
            sc = jnp.dot(jnp.concatenate([q16 * k16, q16 * kf16], axis=0), g4,
                         preferred_element_type=jnp.float32)
            mix1 = jax.nn.sigmoid((sc[16:32] - sc[0:16]) * inv).astype(BF)
            y16 = v16 + mix1 * (kf16 - v16)
            for b in range(4):
                row = jnp.concatenate(
                    [y16[b * 4 + t2:b * 4 + t2 + 1, :] for t2 in range(4)],
                    axis=1)
                ys[b].append(row)
        u_sc[...] = U
        w_sc[...] = W
        for b in range(4):
            y_sc[b, pl.ds(base, UN), :] = jnp.concatenate(ys[b], axis=0)
        return ()

    jax.lax.fori_loop(0, CH // UN, sub, ())

    for b in range(4):
        out_ref[b] = _dot_t(y_sc[b], wo_ref[...])


def kernel(hidden_states, m_gate, alpha_scale, Wq, Wk, Wv, Wo, Wa, ba, mix_logit):
    B, T, D = hidden_states.shape
    BT = B * T
    x = hidden_states.reshape(BT, D)
    gs = (m_gate[..., None] * alpha_scale).reshape(BT, N_HEADS)
    ba2 = ba.reshape(1, N_HEADS)

    nb = BT // TP
    q, k, v, al = pl.pallas_call(
        _proj_kernel,
        grid=(nb,),
        in_specs=[
            pl.BlockSpec((TP, D), lambda i: (i, 0)),
            pl.BlockSpec((D, D), lambda i: (0, 0)),
            pl.BlockSpec((D, D), lambda i: (0, 0)),
            pl.BlockSpec((D, D), lambda i: (0, 0)),
            pl.BlockSpec((N_HEADS, D), lambda i: (0, 0)),
            pl.BlockSpec((1, N_HEADS), lambda i: (0, 0)),
            pl.BlockSpec((TP, N_HEADS), lambda i: (i, 0)),
        ],
        out_specs=[
            pl.BlockSpec((TP, D), lambda i: (i, 0)),
            pl.BlockSpec((TP, D), lambda i: (i, 0)),
            pl.BlockSpec((TP, D), lambda i: (i, 0)),
            pl.BlockSpec((TP, N_HEADS), lambda i: (i, 0)),
        ],
        out_shape=[
            jax.ShapeDtypeStruct((BT, D), BF),
            jax.ShapeDtypeStruct((BT, D), BF),
            jax.ShapeDtypeStruct((BT, D), BF),
            jax.ShapeDtypeStruct((BT, N_HEADS), jnp.float32),
        ],
        compiler_params=pltpu.CompilerParams(
            dimension_semantics=("arbitrary",),
        ),
        name="qkv_alpha_proj",
    )(x, Wq, Wk, Wv, Wa, ba2, gs)

    q4 = q.reshape(B, T, D)
    k4 = k.reshape(B, T, D)
    v4 = v.reshape(B, T, D)
    asel = al.reshape(B, T, 4, 4).transpose(1, 0, 2, 3).reshape(T * 16, 4)

    s4 = jnp.repeat(jnp.eye(4, dtype=jnp.float32), D_HEAD, axis=1)
    g4 = jnp.kron(jnp.eye(4, dtype=BF),
                  jnp.ones((D_HEAD, D_HEAD), BF))

    out = pl.pallas_call(
        _scan_kernel,
        grid=(T // CH,),
        in_specs=[
            pl.BlockSpec((B, CH, D), lambda t: (0, t, 0)),
            pl.BlockSpec((B, CH, D), lambda t: (0, t, 0)),
            pl.BlockSpec((B, CH, D), lambda t: (0, t, 0)),
            pl.BlockSpec((CH * 16, 4), lambda t: (t, 0)),
            pl.BlockSpec((4, 256), lambda t: (0, 0)),
            pl.BlockSpec((256, 256), lambda t: (0, 0)),
            pl.BlockSpec((D, D), lambda t: (0, 0)),
        ],
        out_specs=pl.BlockSpec((B, CH, D), lambda t: (0, t, 0)),
        out_shape=jax.ShapeDtypeStruct((B, T, D), jnp.float32),
        scratch_shapes=[
            pltpu.VMEM((B, CH, D), BF),
            pltpu.VMEM((CH, 16, 256), BF),
            pltpu.VMEM((256, 256), BF),
            pltpu.VMEM((256, 256), BF),
        ],
        compiler_params=pltpu.CompilerParams(
            dimension_semantics=("arbitrary",),
        ),
        name="fast_weight_scan",
    )(q4, k4, v4, asel, s4, g4, Wo)

    return out


# speedup vs baseline: 1.1178x; 1.0068x over previous
import jax
import jax.numpy as jnp
from jax.experimental import pallas as pl
from jax.experimental.pallas import tpu as pltpu

D_MODEL = 1024
N_HEADS = 16
D_HEAD = 64
RANK = 16
DECAY = 0.95
ALPHA_MAX = 0.05
BETA = 0.01

TP = 256
CH = 256
UN = 64

BF = jnp.bfloat16


def _dot_t(a, b):
    return jax.lax.dot_general(a, b, (((1,), (1,)), ((), ())),
                               preferred_element_type=jnp.float32)


def _proj_kernel(x_ref, wq_ref, wk_ref, wv_ref, wa_ref, ba_ref, gs_ref,
                 q_ref, k_ref, v_ref, al_ref):
    x = x_ref[...]
    q_ref[...] = _dot_t(x, wq_ref[...]).astype(BF)
    k_ref[...] = _dot_t(x, wk_ref[...]).astype(BF)
    v_ref[...] = _dot_t(x, wv_ref[...]).astype(BF)
    aa = _dot_t(x, wa_ref[...]) + ba_ref[...]
    al = jax.nn.sigmoid(aa) * gs_ref[...]
    al_ref[...] = jnp.minimum(al, ALPHA_MAX)


def _make16(rows):
    pieces = []
    for r in rows:
        pieces.extend([r[:, 256 * t2:256 * (t2 + 1)] for t2 in range(4)])
    return jnp.concatenate(pieces, axis=0)


def _softdecay(x):
    return x - BF(1.0 - DECAY) * x


def _scan_kernel(q_ref, k_ref, v_ref, asel_ref, s4_ref, g4_ref, wo_ref,
                 out_ref, y_sc, adt_sc, u_sc, w_sc):
    jc = pl.program_id(0)

    @pl.when(jc == 0)
    def _():
        u_sc[...] = jnp.zeros_like(u_sc)
        w_sc[...] = jnp.zeros_like(w_sc)

    adT = jnp.dot(asel_ref[...], s4_ref[...],
                  preferred_element_type=jnp.float32)
    adt_sc[...] = adT.astype(BF).reshape(CH, 16, 256)
    g4 = g4_ref[...]
    inv = 0.125
    beta = BF(BETA)
    one = BF(1.0)

    def sub(ci, _):
        U = u_sc[...]
        W = w_sc[...]
        base = pl.multiple_of(ci * UN, UN)
        k_sl = [k_ref[b, pl.ds(base, UN), :] for b in range(4)]
        q_sl = [q_ref[b, pl.ds(base, UN), :] for b in range(4)]
        v_sl = [v_ref[b, pl.ds(base, UN), :] for b in range(4)]
        for ii in range(UN):
            t = base + ii
            k16 = _make16([k_sl[b][ii:ii + 1, :] for b in range(4)])
            q16 = _make16([q_sl[b][ii:ii + 1, :] for b in range(4)])
            v16 = _make16([v_sl[b][ii:ii + 1, :] for b in range(4)])
            a16 = adt_sc[pl.ds(t, 1)].reshape(16, 256)
            k256 = pltpu.repeat(k16, 16, axis=0)
            P = k256 * U
            kub = jnp.concatenate(
                [jnp.dot(P[0:128], g4,
                         preferred_element_type=jnp.float32).astype(BF),
                 jnp.dot(P[128:256], g4,
                         preferred_element_type=jnp.float32).astype(BF)],
                axis=0)
            tm = pltpu.repeat(a16, 16, axis=0) * kub
            Un = _softdecay(U + k256 * tm)
            Wn = _softdecay(W + pltpu.repeat(v16, 16, axis=0) * tm)
            U = Un - beta * jnp.clip(Un, -one, one)
            W = Wn - beta * jnp.clip(Wn, -one, one)
            z = U * W
            zs = z[0:128] + z[128:256]
            zs = zs[0:64] + zs[64:128]
            zs = zs[0:32] + zs[32:64]
            kf16 = zs[0:16] + zs[16:32]
            sc = jnp.dot(jnp.concatenate([q16 * k16, q16 * kf16], axis=0), g4,
                         preferred_element_type=jnp.float32)
            mix1 = jax.nn.sigmoid((sc[16:32] - sc[0:16]) * inv).astype(BF)
            y16 = v16 + mix1 * (kf16 - v16)
            y_sc[pl.ds(t, 1)] = y16.reshape(1, 16, 256)
        u_sc[...] = U
        w_sc[...] = W
        return ()

    jax.lax.fori_loop(0, CH // UN, sub, ())

    wo = wo_ref[...]
    for b in range(4):
        acc = _dot_t(y_sc[:, 4 * b, :], wo[:, 0:256])
        for jj in range(1, 4):
            acc += _dot_t(y_sc[:, 4 * b + jj, :],
                          wo[:, 256 * jj:256 * (jj + 1)])
        out_ref[b] = acc


def kernel(hidden_states, m_gate, alpha_scale, Wq, Wk, Wv, Wo, Wa, ba, mix_logit):
    B, T, D = hidden_states.shape
    BT = B * T
    x = hidden_states.reshape(BT, D)
    gs = (m_gate[..., None] * alpha_scale).reshape(BT, N_HEADS)
    ba2 = ba.reshape(1, N_HEADS)

    nb = BT // TP
    q, k, v, al = pl.pallas_call(
        _proj_kernel,
        grid=(nb,),
        in_specs=[
            pl.BlockSpec((TP, D), lambda i: (i, 0)),
            pl.BlockSpec((D, D), lambda i: (0, 0)),
            pl.BlockSpec((D, D), lambda i: (0, 0)),
            pl.BlockSpec((D, D), lambda i: (0, 0)),
            pl.BlockSpec((N_HEADS, D), lambda i: (0, 0)),
            pl.BlockSpec((1, N_HEADS), lambda i: (0, 0)),
            pl.BlockSpec((TP, N_HEADS), lambda i: (i, 0)),
        ],
        out_specs=[
            pl.BlockSpec((TP, D), lambda i: (i, 0)),
            pl.BlockSpec((TP, D), lambda i: (i, 0)),
            pl.BlockSpec((TP, D), lambda i: (i, 0)),
            pl.BlockSpec((TP, N_HEADS), lambda i: (i, 0)),
        ],
        out_shape=[
            jax.ShapeDtypeStruct((BT, D), BF),
            jax.ShapeDtypeStruct((BT, D), BF),
            jax.ShapeDtypeStruct((BT, D), BF),
            jax.ShapeDtypeStruct((BT, N_HEADS), jnp.float32),
        ],
        compiler_params=pltpu.CompilerParams(
            dimension_semantics=("arbitrary",),
        ),
        name="qkv_alpha_proj",
    )(x, Wq, Wk, Wv, Wa, ba2, gs)

    q4 = q.reshape(B, T, D)
    k4 = k.reshape(B, T, D)
    v4 = v.reshape(B, T, D)
    asel = al.reshape(B, T, 4, 4).transpose(1, 0, 2, 3).reshape(T * 16, 4)

    s4 = jnp.repeat(jnp.eye(4, dtype=jnp.float32), D_HEAD, axis=1)
    g4 = jnp.kron(jnp.eye(4, dtype=BF),
                  jnp.ones((D_HEAD, D_HEAD), BF))

    out = pl.pallas_call(
        _scan_kernel,
        grid=(T // CH,),
        in_specs=[
            pl.BlockSpec((B, CH, D), lambda t: (0, t, 0)),
            pl.BlockSpec((B, CH, D), lambda t: (0, t, 0)),
            pl.BlockSpec((B, CH, D), lambda t: (0, t, 0)),
            pl.BlockSpec((CH * 16, 4), lambda t: (t, 0)),
            pl.BlockSpec((4, 256), lambda t: (0, 0)),
            pl.BlockSpec((256, 256), lambda t: (0, 0)),
            pl.BlockSpec((D, D), lambda t: (0, 0)),
        ],
        out_specs=pl.BlockSpec((B, CH, D), lambda t: (0, t, 0)),
        out_shape=jax.ShapeDtypeStruct((B, T, D), jnp.float32),
        scratch_shapes=[
            pltpu.VMEM((CH, 16, 256), BF),
            pltpu.VMEM((CH, 16, 256), BF),
            pltpu.VMEM((256, 256), BF),
            pltpu.VMEM((256, 256), BF),
        ],
        compiler_params=pltpu.CompilerParams(
            dimension_semantics=("arbitrary",),
        ),
        name="fast_weight_scan",
    )(q4, k4, v4, asel, s4, g4, Wo)

    return out
